# baseline (speedup 1.0000x reference)
import functools

import jax
import jax.numpy as jnp
from jax import lax
from jax.experimental import pallas as pl
from jax.experimental.pallas import tpu as pltpu
from jax.experimental.pallas import tpu_sc as plsc

VOCAB = 1000000
EMBED = 32
B, L = 4096, 200
N = B * L

NC, NS, LANES = 2, 16, 16
NW = NC * NS
RPB = 128
N_LT = L // 8
N_BT = B // RPB
N_ET = EMBED // 8
BB = RPB // LANES
NU = L
DEPTH = 4


def _sc_embed(tok5, val5, tt, vt):
    mesh = plsc.VectorSubcoreMesh(core_axis_name="c", subcore_axis_name="s")

    @functools.partial(
        pl.kernel,
        mesh=mesh,
        compiler_params=pltpu.CompilerParams(
            use_tc_tiling_on_sc=False, needs_layout_passes=False,
            disable_bounds_checks=True),
        out_type=jax.ShapeDtypeStruct((L, N_ET, N_BT, 8, RPB), jnp.float32),
        scratch_types=[
            pltpu.VMEM((2, 8, RPB), jnp.int32),
            pltpu.VMEM((2, 8, RPB), jnp.float32),
            pltpu.VMEM((DEPTH, RPB, EMBED), jnp.float32),
            pltpu.VMEM((DEPTH, RPB, EMBED), jnp.float32),
            pltpu.VMEM((DEPTH, EMBED, RPB), jnp.float32),
            pltpu.SemaphoreType.DMA((DEPTH,)),
            pltpu.SemaphoreType.DMA((DEPTH,)),
            pltpu.SemaphoreType.DMA,
        ],
    )
    def k(tok_hbm, val_hbm, tt_hbm, vt_hbm, out_hbm,
          idx_v, vals_v, trows, vrows, outT, gsem, osem, ssem):
        cid = lax.axis_index("c")
        sid = lax.axis_index("s")
        w = sid * NC + cid

        iota = lax.iota(jnp.int32, LANES)

        def fire_gather(u):
            bf = u % DEPTH
            lt2 = (u // 8) % 2
            li = u % 8
            pltpu.async_copy(tt_hbm.at[idx_v.at[lt2, li]], trows.at[bf], gsem.at[bf])
            pltpu.async_copy(vt_hbm.at[idx_v.at[lt2, li]], vrows.at[bf], gsem.at[bf])

        def wait_gather(u):
            bf = u % DEPTH
            pltpu.make_async_copy(tt_hbm.at[pl.ds(0, RPB)], trows.at[bf], gsem.at[bf]).wait()
            pltpu.make_async_copy(vt_hbm.at[pl.ds(0, RPB)], vrows.at[bf], gsem.at[bf]).wait()

        def drain_out(u):
            bf = u % DEPTH
            for et in range(N_ET):
                pltpu.make_async_copy(
                    outT.at[bf, pl.ds(et * 8, 8)], out_hbm.at[u, et, w], osem.at[bf]).wait()

        pltpu.sync_copy(tok_hbm.at[0, w], idx_v.at[0])
        pltpu.sync_copy(val_hbm.at[0, w], vals_v.at[0])
        for up in range(DEPTH - 1):
            fire_gather(up)

        def unit_body(u, carry):
            lt = u // 8
            li = u % 8
            lt2 = lt % 2
            bf = u % DEPTH

            @pl.when(jnp.logical_and(li == 0, lt < N_LT - 1))
            def _():
                pltpu.async_copy(tok_hbm.at[lt + 1, w], idx_v.at[(lt + 1) % 2], ssem)
                pltpu.async_copy(val_hbm.at[lt + 1, w], vals_v.at[(lt + 1) % 2], ssem)

            @pl.when(jnp.logical_and(li == 8 - (DEPTH - 1), lt < N_LT - 1))
            def _():
                pltpu.make_async_copy(tok_hbm.at[0, w], idx_v.at[0], ssem).wait()
                pltpu.make_async_copy(val_hbm.at[0, w], vals_v.at[0], ssem).wait()

            @pl.when(u < NU - (DEPTH - 1))
            def _():
                fire_gather(u + DEPTH - 1)

            wait_gather(u)

            @pl.when(u >= DEPTH)
            def _():
                drain_out(u - DEPTH)

            for bb in range(BB):
                tokv = idx_v[lt2, li, pl.ds(bb * LANES, LANES)]
                valv = vals_v[lt2, li, pl.ds(bb * LANES, LANES)]
                nz = tokv != 0
                m = jnp.where(nz, 1.0, 0.0)
                wv = jnp.where(nz & (valv == valv), valv, 0.0)
                row_idx = iota + (bb * LANES)
                for e in range(EMBED):
                    col_idx = jnp.full((LANES,), e, jnp.int32)
                    t = plsc.load_gather(trows.at[bf], [row_idx, col_idx])
                    v = plsc.load_gather(vrows.at[bf], [row_idx, col_idx])
                    outT[bf, e, pl.ds(bb * LANES, LANES)] = t * m + v * wv

            for et in range(N_ET):
                pltpu.async_copy(
                    outT.at[bf, pl.ds(et * 8, 8)], out_hbm.at[u, et, w], osem.at[bf])
            return carry

        lax.fori_loop(0, NU, unit_body, 0)
        for ue in range(NU - DEPTH, NU):
            drain_out(ue)

    return k(tok5, val5, tt, vt)


def kernel(tokens, values, token_table, value_table):
    tok5 = tokens.T.reshape(N_LT, 8, N_BT, RPB).transpose(0, 2, 1, 3)
    val5 = values.T.reshape(N_LT, 8, N_BT, RPB).transpose(0, 2, 1, 3)
    out5 = _sc_embed(tok5, val5, token_table, value_table)
    return out5.transpose(2, 4, 0, 1, 3).reshape(B, L, EMBED)


# speedup vs baseline: 1.4670x; 1.4670x over previous
import functools

import jax
import jax.numpy as jnp
from jax import lax
from jax.experimental import pallas as pl
from jax.experimental.pallas import tpu as pltpu
from jax.experimental.pallas import tpu_sc as plsc

VOCAB = 1000000
EMBED = 32
B, L = 4096, 200
N = B * L

NC, NS, LANES = 2, 16, 16
NW = NC * NS
RPB = 128
N_LT = L // 8
N_BT = B // RPB
N_ET = EMBED // 8
BB = RPB // LANES
NU = L
DEPTH = 4


def _sc_embed(tok5, val5, tt, vt):
    mesh = plsc.VectorSubcoreMesh(core_axis_name="c", subcore_axis_name="s")

    @functools.partial(
        pl.kernel,
        mesh=mesh,
        compiler_params=pltpu.CompilerParams(
            use_tc_tiling_on_sc=False, needs_layout_passes=False,
            disable_bounds_checks=True),
        out_type=jax.ShapeDtypeStruct((L, N_ET, N_BT, 8, RPB), jnp.float32),
        scratch_types=[
            pltpu.VMEM((2, 8, RPB), jnp.int32),
            pltpu.VMEM((2, 8, RPB), jnp.float32),
            pltpu.VMEM((DEPTH, RPB, EMBED), jnp.float32),
            pltpu.VMEM((DEPTH, RPB, EMBED), jnp.float32),
            pltpu.VMEM((DEPTH, EMBED, RPB), jnp.float32),
            pltpu.SemaphoreType.DMA((DEPTH,)),
            pltpu.SemaphoreType.DMA((DEPTH,)),
            pltpu.SemaphoreType.DMA,
        ],
    )
    def k(tok_hbm, val_hbm, tt_hbm, vt_hbm, out_hbm,
          idx_v, vals_v, trows, vrows, outT, gsem, osem, ssem):
        cid = lax.axis_index("c")
        sid = lax.axis_index("s")
        w = sid * NC + cid

        iota = lax.iota(jnp.int32, LANES)

        def fire_gather(u):
            bf = u % DEPTH
            lt2 = (u // 8) % 2
            li = u % 8
            pltpu.async_copy(tt_hbm.at[idx_v.at[lt2, li]], trows.at[bf], gsem.at[bf])
            pltpu.async_copy(vt_hbm.at[idx_v.at[lt2, li]], vrows.at[bf], gsem.at[bf])

        def wait_gather(u):
            bf = u % DEPTH
            pltpu.make_async_copy(tt_hbm.at[pl.ds(0, RPB)], trows.at[bf], gsem.at[bf]).wait()
            pltpu.make_async_copy(vt_hbm.at[pl.ds(0, RPB)], vrows.at[bf], gsem.at[bf]).wait()

        def drain_out(u):
            bf = u % DEPTH
            for et in range(N_ET):
                pltpu.make_async_copy(
                    outT.at[bf, pl.ds(et * 8, 8)], out_hbm.at[u, et, w], osem.at[bf]).wait()

        pltpu.sync_copy(tok_hbm.at[0, w], idx_v.at[0])
        pltpu.sync_copy(val_hbm.at[0, w], vals_v.at[0])
        for up in range(DEPTH - 1):
            fire_gather(up)

        def unit_body(u, carry):
            lt = u // 8
            li = u % 8
            lt2 = lt % 2
            bf = u % DEPTH

            @pl.when(jnp.logical_and(li == 0, lt < N_LT - 1))
            def _():
                pltpu.async_copy(tok_hbm.at[lt + 1, w], idx_v.at[(lt + 1) % 2], ssem)
                pltpu.async_copy(val_hbm.at[lt + 1, w], vals_v.at[(lt + 1) % 2], ssem)

            @pl.when(jnp.logical_and(li == 8 - (DEPTH - 1), lt < N_LT - 1))
            def _():
                pltpu.make_async_copy(tok_hbm.at[0, w], idx_v.at[0], ssem).wait()
                pltpu.make_async_copy(val_hbm.at[0, w], vals_v.at[0], ssem).wait()

            @pl.when(u < NU - (DEPTH - 1))
            def _():
                fire_gather(u + DEPTH - 1)

            wait_gather(u)

            @pl.when(u >= DEPTH)
            def _():
                drain_out(u - DEPTH)

            for bb in range(BB):
                tokv = idx_v[lt2, li, pl.ds(bb * LANES, LANES)]
                valv = vals_v[lt2, li, pl.ds(bb * LANES, LANES)]
                nz = tokv != 0
                m = jnp.where(nz, 1.0, 0.0)
                wv = jnp.where(nz & (valv == valv), valv, 0.0)
                row_idx = iota + (bb * LANES)
                for e0 in range(EMBED):
                    col_idx = (iota + e0) % EMBED
                    t = plsc.load_gather(trows.at[bf], [row_idx, col_idx])
                    v = plsc.load_gather(vrows.at[bf], [row_idx, col_idx])
                    plsc.store_scatter(outT.at[bf], [col_idx, row_idx], t * m + v * wv)

            for et in range(N_ET):
                pltpu.async_copy(
                    outT.at[bf, pl.ds(et * 8, 8)], out_hbm.at[u, et, w], osem.at[bf])
            return carry

        lax.fori_loop(0, NU, unit_body, 0)
        for ue in range(NU - DEPTH, NU):
            drain_out(ue)

    return k(tok5, val5, tt, vt)


def kernel(tokens, values, token_table, value_table):
    tok5 = tokens.T.reshape(N_LT, 8, N_BT, RPB).transpose(0, 2, 1, 3)
    val5 = values.T.reshape(N_LT, 8, N_BT, RPB).transpose(0, 2, 1, 3)
    out5 = _sc_embed(tok5, val5, token_table, value_table)
    return out5.transpose(2, 4, 0, 1, 3).reshape(B, L, EMBED)


# speedup vs baseline: 1.7001x; 1.1589x over previous
import functools

import jax
import jax.numpy as jnp
from jax import lax
from jax.experimental import pallas as pl
from jax.experimental.pallas import tpu as pltpu
from jax.experimental.pallas import tpu_sc as plsc

VOCAB = 1000000
VPAD = 1000064
EMBED = 32
B, L = 4096, 200
N = B * L

NC, NS, LANES = 2, 16, 16
NW = NC * NS
RPB = 128
N_LT = L // 8
N_BT = B // RPB
N_ET = EMBED // 8
BB = RPB // LANES
NU = L
DEPTH = 4


def _sc_table_relayout(ttT, vtT):
    mesh = plsc.VectorSubcoreMesh(core_axis_name="c", subcore_axis_name="s")
    NTC = VPAD // RPB

    @functools.partial(
        pl.kernel,
        mesh=mesh,
        compiler_params=pltpu.CompilerParams(
            use_tc_tiling_on_sc=True, needs_layout_passes=False,
            disable_bounds_checks=True),
        out_type=(jax.ShapeDtypeStruct((VPAD * EMBED,), jnp.float32),
                  jax.ShapeDtypeStruct((VPAD * EMBED,), jnp.float32)),
        scratch_types=[
            pltpu.VMEM((2 * 4 * 16, RPB), jnp.float32),
            pltpu.VMEM((2 * 2 * RPB * EMBED,), jnp.float32),
            pltpu.SemaphoreType.DMA((2,)),
            pltpu.SemaphoreType.DMA((2,)),
        ],
    )
    def k(ttT_hbm, vtT_hbm, tt_out, vt_out, blk, outb, rsem, osem):
        cid = lax.axis_index("c")
        sid = lax.axis_index("s")
        w = sid * NC + cid
        nb = jnp.where(w < NTC % NW, NTC // NW + 1, NTC // NW)
        iota = lax.iota(jnp.int32, LANES)
        srcs = (ttT_hbm, vtT_hbm)
        outs = (tt_out, vt_out)

        def fire_read(kk):
            bf = kk % 2
            tc = w + kk * NW
            for tbl in range(2):
                for et2 in range(2):
                    pltpu.async_copy(
                        srcs[tbl].at[pl.ds(et2 * 16, 16), pl.ds(tc * RPB, RPB)],
                        blk.at[pl.ds((bf * 4 + tbl * 2 + et2) * 16, 16)],
                        rsem.at[bf])

        def wait_read(kk):
            bf = kk % 2
            for j in range(4):
                pltpu.make_async_copy(
                    srcs[0].at[pl.ds(0, 16), pl.ds(0, RPB)],
                    blk.at[pl.ds((bf * 4 + j) * 16, 16)], rsem.at[bf]).wait()

        def drain_write(kk):
            bf = kk % 2
            tc = kk
            for tbl in range(2):
                pltpu.make_async_copy(
                    outb.at[pl.ds((bf * 2 + tbl) * RPB * EMBED, RPB * EMBED)],
                    outs[tbl].at[pl.ds(0, RPB * EMBED)], osem.at[bf]).wait()

        fire_read(0)

        def body(kk, carry):
            bf = kk % 2
            tc = w + kk * NW

            @pl.when(kk + 1 < nb)
            def _():
                fire_read(kk + 1)

            wait_read(kk)

            @pl.when(kk >= 2)
            def _():
                drain_write(kk - 2)

            for tbl in range(2):
                for et2 in range(2):
                    b2 = blk.at[pl.ds((bf * 4 + tbl * 2 + et2) * 16, 16)]
                    ob_base = (bf * 2 + tbl) * (RPB * EMBED)
                    for tl in range(8):
                        t_vec = iota + tl * 16
                        ob_t = ob_base + t_vec * EMBED + (et2 * 16)

                        def s_body(s, c3, b2=b2, t_vec=t_vec, ob_t=ob_t):
                            e_vec = (iota + s) & 15
                            x = plsc.load_gather(b2, [e_vec, t_vec])
                            plsc.store_scatter(outb, [ob_t + e_vec], x)
                            return c3

                        lax.fori_loop(0, 16, s_body, 0)
            for tbl in range(2):
                pltpu.async_copy(
                    outb.at[pl.ds((bf * 2 + tbl) * RPB * EMBED, RPB * EMBED)],
                    outs[tbl].at[pl.ds(tc * (RPB * EMBED), RPB * EMBED)],
                    osem.at[bf])
            return carry

        lax.fori_loop(0, nb, body, 0)

        @pl.when(nb >= 2)
        def _():
            drain_write(nb - 2)

        @pl.when(nb >= 1)
        def _():
            drain_write(nb - 1)

    return k(ttT, vtT)


def _sc_embed(tok5, val5, tt, vt):
    mesh = plsc.VectorSubcoreMesh(core_axis_name="c", subcore_axis_name="s")

    @functools.partial(
        pl.kernel,
        mesh=mesh,
        compiler_params=pltpu.CompilerParams(
            use_tc_tiling_on_sc=False, needs_layout_passes=False,
            disable_bounds_checks=True),
        out_type=jax.ShapeDtypeStruct((L, N_ET, N_BT, 8, RPB), jnp.float32),
        scratch_types=[
            pltpu.VMEM((2, 8, RPB), jnp.int32),
            pltpu.VMEM((2, 8, RPB), jnp.float32),
            pltpu.VMEM((DEPTH, RPB, EMBED), jnp.float32),
            pltpu.VMEM((DEPTH, RPB, EMBED), jnp.float32),
            pltpu.VMEM((DEPTH, EMBED, RPB), jnp.float32),
            pltpu.SemaphoreType.DMA((DEPTH,)),
            pltpu.SemaphoreType.DMA((DEPTH,)),
            pltpu.SemaphoreType.DMA,
        ],
    )
    def k(tok_hbm, val_hbm, tt_hbm, vt_hbm, out_hbm,
          idx_v, vals_v, trows, vrows, outT, gsem, osem, ssem):
        cid = lax.axis_index("c")
        sid = lax.axis_index("s")
        w = sid * NC + cid

        iota = lax.iota(jnp.int32, LANES)

        def fire_gather(u):
            bf = u % DEPTH
            lt2 = (u // 8) % 2
            li = u % 8
            pltpu.async_copy(tt_hbm.at[idx_v.at[lt2, li]], trows.at[bf], gsem.at[bf])
            pltpu.async_copy(vt_hbm.at[idx_v.at[lt2, li]], vrows.at[bf], gsem.at[bf])

        def wait_gather(u):
            bf = u % DEPTH
            pltpu.make_async_copy(tt_hbm.at[pl.ds(0, RPB)], trows.at[bf], gsem.at[bf]).wait()
            pltpu.make_async_copy(vt_hbm.at[pl.ds(0, RPB)], vrows.at[bf], gsem.at[bf]).wait()

        def drain_out(u):
            bf = u % DEPTH
            for et in range(N_ET):
                pltpu.make_async_copy(
                    outT.at[bf, pl.ds(et * 8, 8)], out_hbm.at[u, et, w], osem.at[bf]).wait()

        pltpu.sync_copy(tok_hbm.at[0, w], idx_v.at[0])
        pltpu.sync_copy(val_hbm.at[0, w], vals_v.at[0])
        for up in range(DEPTH - 1):
            fire_gather(up)

        def unit_body(u, carry):
            lt = u // 8
            li = u % 8
            lt2 = lt % 2
            bf = u % DEPTH

            @pl.when(jnp.logical_and(li == 0, lt < N_LT - 1))
            def _():
                pltpu.async_copy(tok_hbm.at[lt + 1, w], idx_v.at[(lt + 1) % 2], ssem)
                pltpu.async_copy(val_hbm.at[lt + 1, w], vals_v.at[(lt + 1) % 2], ssem)

            @pl.when(jnp.logical_and(li == 8 - (DEPTH - 1), lt < N_LT - 1))
            def _():
                pltpu.make_async_copy(tok_hbm.at[0, w], idx_v.at[0], ssem).wait()
                pltpu.make_async_copy(val_hbm.at[0, w], vals_v.at[0], ssem).wait()

            @pl.when(u < NU - (DEPTH - 1))
            def _():
                fire_gather(u + DEPTH - 1)

            wait_gather(u)

            @pl.when(u >= DEPTH)
            def _():
                drain_out(u - DEPTH)

            for bb in range(BB):
                tokv = idx_v[lt2, li, pl.ds(bb * LANES, LANES)]
                valv = vals_v[lt2, li, pl.ds(bb * LANES, LANES)]
                nz = tokv != 0
                m = jnp.where(nz, 1.0, 0.0)
                wv = jnp.where(nz & (valv == valv), valv, 0.0)
                row_idx = iota + (bb * LANES)
                for e0 in range(EMBED):
                    col_idx = (iota + e0) % EMBED
                    t = plsc.load_gather(trows.at[bf], [row_idx, col_idx])
                    v = plsc.load_gather(vrows.at[bf], [row_idx, col_idx])
                    plsc.store_scatter(outT.at[bf], [col_idx, row_idx], t * m + v * wv)

            for et in range(N_ET):
                pltpu.async_copy(
                    outT.at[bf, pl.ds(et * 8, 8)], out_hbm.at[u, et, w], osem.at[bf])
            return carry

        lax.fori_loop(0, NU, unit_body, 0)
        for ue in range(NU - DEPTH, NU):
            drain_out(ue)

    return k(tok5, val5, tt, vt)


def kernel(tokens, values, token_table, value_table):
    tt_lin, vt_lin = _sc_table_relayout(token_table.T, value_table.T)
    tt2 = tt_lin.reshape(VPAD, EMBED)
    vt2 = vt_lin.reshape(VPAD, EMBED)
    tok5 = tokens.T.reshape(N_LT, 8, N_BT, RPB).transpose(0, 2, 1, 3)
    val5 = values.T.reshape(N_LT, 8, N_BT, RPB).transpose(0, 2, 1, 3)
    out5 = _sc_embed(tok5, val5, tt2, vt2)
    return out5.transpose(2, 4, 0, 1, 3).reshape(B, L, EMBED)
